# Initial kernel scaffold; baseline (speedup 1.0000x reference)
import jax
import jax.numpy as jnp
from jax import lax
from jax.experimental import pallas as pl
from jax.experimental.pallas import tpu as pltpu
from jax.experimental.pallas import tpu_sc as plsc

C = 160
NPTS = 500
NP_NEIGH = 8
LANES = 16
PADP = 512
NW = 32
CPW = C // NW
NBLK = PADP // LANES


def _sc_body(f1_hbm, f2_hbm, inds_hbm, out_hbm, indsv, f1v, f2v, outv):
    wid = lax.axis_index("s") * 2 + lax.axis_index("c")
    c0 = wid * CPW
    pltpu.sync_copy(inds_hbm, indsv)
    pltpu.sync_copy(f1_hbm.at[pl.ds(c0, CPW)], f1v)
    pltpu.sync_copy(f2_hbm.at[pl.ds(c0, CPW)], f2v)

    def block(b, carry):
        off = b * LANES
        accs = [jnp.zeros((LANES,), jnp.float32) for _ in range(CPW)]
        for j in range(NP_NEIGH):
            gidx = indsv[j, pl.ds(off, LANES)]
            for k in range(CPW):
                row = jnp.full((LANES,), k, jnp.int32)
                accs[k] = accs[k] + plsc.load_gather(f2v, [row, gidx])
        scale = 1.0 / NP_NEIGH
        for k in range(CPW):
            outv[k, pl.ds(off, LANES)] = (
                accs[k] * f1v[k, pl.ds(off, LANES)] * scale)
        return carry

    lax.fori_loop(0, NBLK, block, 0)
    pltpu.sync_copy(outv, out_hbm.at[pl.ds(c0, CPW)])


def kernel(feat1, feat2, inds):
    f1p = jnp.zeros((C, PADP), jnp.float32).at[:, :NPTS].set(feat1[0])
    f2p = jnp.zeros((C, PADP), jnp.float32).at[:, :NPTS].set(feat2[0])
    inds_t = (jnp.zeros((NP_NEIGH, PADP), jnp.int32)
              .at[:, :NPTS].set(inds[0].astype(jnp.int32).T))

    run = pl.kernel(
        _sc_body,
        mesh=plsc.VectorSubcoreMesh(core_axis_name="c", subcore_axis_name="s"),
        out_type=jax.ShapeDtypeStruct((C, PADP), jnp.float32),
        scratch_types=[
            pltpu.VMEM((NP_NEIGH, PADP), jnp.int32),
            pltpu.VMEM((CPW, PADP), jnp.float32),
            pltpu.VMEM((CPW, PADP), jnp.float32),
            pltpu.VMEM((CPW, PADP), jnp.float32),
        ],
    )
    outp = run(f1p, f2p, inds_t)
    return outp[:, :NPTS].reshape(1, C, NPTS)


# speedup vs baseline: 219.0888x; 219.0888x over previous
import jax
import jax.numpy as jnp
from jax import lax
from jax.experimental import pallas as pl
from jax.experimental.pallas import tpu as pltpu
from jax.experimental.pallas import tpu_sc as plsc

C = 160
NPTS = 500
NP_NEIGH = 8
LANES = 16
PADP = 512
NW = 32
CPW = C // NW
NBLK = PADP // LANES


def _sc_body(f1_hbm, f2_hbm, inds_hbm, out_hbm, indsv, f1v, f2v, outv):
    wid = lax.axis_index("s") * 2 + lax.axis_index("c")
    c0 = wid * CPW
    pltpu.sync_copy(inds_hbm, indsv)
    pltpu.sync_copy(f1_hbm.at[pl.ds(c0, CPW)], f1v)
    pltpu.sync_copy(f2_hbm.at[pl.ds(c0, CPW)], f2v)

    def block(b, carry):
        off = b * LANES
        accs = [jnp.zeros((LANES,), jnp.float32) for _ in range(CPW)]
        for j in range(NP_NEIGH):
            gidx = indsv[j, pl.ds(off, LANES)]
            for k in range(CPW):
                row = jnp.full((LANES,), k, jnp.int32)
                accs[k] = accs[k] + plsc.load_gather(f2v, [row, gidx])
        scale = 1.0 / NP_NEIGH
        for k in range(CPW):
            outv[k, pl.ds(off, LANES)] = (
                accs[k] * f1v[k, pl.ds(off, LANES)] * scale)
        return carry

    lax.fori_loop(0, NBLK, block, 0)
    pltpu.sync_copy(outv, out_hbm.at[pl.ds(c0, CPW)])


def kernel(feat1, feat2, inds):
    f1p = jnp.zeros((C, PADP), jnp.float32).at[:, :NPTS].set(feat1[0])
    f2p = jnp.zeros((C, PADP), jnp.float32).at[:, :NPTS].set(feat2[0])
    inds_t = (jnp.zeros((NP_NEIGH, PADP), jnp.int32)
              .at[:, :NPTS].set(inds[0].astype(jnp.int32).T))

    run = pl.kernel(
        _sc_body,
        mesh=plsc.VectorSubcoreMesh(core_axis_name="c", subcore_axis_name="s"),
        compiler_params=pltpu.CompilerParams(use_tc_tiling_on_sc=False,
                                             needs_layout_passes=False),
        out_type=jax.ShapeDtypeStruct((C, PADP), jnp.float32),
        scratch_types=[
            pltpu.VMEM((NP_NEIGH, PADP), jnp.int32),
            pltpu.VMEM((CPW, PADP), jnp.float32),
            pltpu.VMEM((CPW, PADP), jnp.float32),
            pltpu.VMEM((CPW, PADP), jnp.float32),
        ],
    )
    outp = run(f1p, f2p, inds_t)
    return outp[:, :NPTS].reshape(1, C, NPTS)


# speedup vs baseline: 238.9977x; 1.0909x over previous
import jax
import jax.numpy as jnp
from jax import lax
from jax.experimental import pallas as pl
from jax.experimental.pallas import tpu as pltpu
from jax.experimental.pallas import tpu_sc as plsc

C = 160
NPTS = 500
NP_NEIGH = 8
LANES = 16
NW = 32
CPW = C // NW
CHUNK = CPW * NPTS
NBLK = 32
TAIL_OFF = NPTS - LANES


def _sc_body(f1_hbm, f2_hbm, inds_hbm, out_hbm, indsv, f1v, f2v, outv,
             sem1, sem2, sem3):
    wid = lax.axis_index("s") * 2 + lax.axis_index("c")
    cp2 = pltpu.async_copy(f2_hbm.at[wid], f2v, sem2)
    cpi = pltpu.async_copy(inds_hbm, indsv, sem3)
    cp1 = pltpu.async_copy(f1_hbm.at[wid], f1v, sem1)
    cp2.wait()
    cpi.wait()

    lanes = lax.iota(jnp.int32, LANES)

    def block(b, carry):
        off = jnp.minimum(b * LANES, TAIL_OFF)
        pos8 = (off + lanes) * NP_NEIGH
        accs = [jnp.zeros((LANES,), jnp.float32) for _ in range(CPW)]
        for j in range(NP_NEIGH):
            gidx = plsc.load_gather(indsv, [pos8 + j])
            for k in range(CPW):
                accs[k] = accs[k] + plsc.load_gather(f2v, [gidx + (k * NPTS)])
        return carry, off, accs

    def block_and_store(b, carry):
        carry, off, accs = block(b, carry)
        scale = 1.0 / NP_NEIGH
        for k in range(CPW):
            outv[pl.ds(k * NPTS + off, LANES)] = (
                accs[k] * f1v[pl.ds(k * NPTS + off, LANES)] * scale)
        return carry

    cp1.wait()
    lax.fori_loop(0, NBLK, block_and_store, 0)
    pltpu.sync_copy(outv, out_hbm.at[wid])


def kernel(feat1, feat2, inds):
    f1 = feat1.reshape(NW, CHUNK)
    f2 = feat2.reshape(NW, CHUNK)
    iflat = inds.astype(jnp.int32).reshape(NPTS * NP_NEIGH)

    run = pl.kernel(
        _sc_body,
        mesh=plsc.VectorSubcoreMesh(core_axis_name="c", subcore_axis_name="s"),
        compiler_params=pltpu.CompilerParams(use_tc_tiling_on_sc=False,
                                             needs_layout_passes=False),
        out_type=jax.ShapeDtypeStruct((NW, CHUNK), jnp.float32),
        scratch_types=[
            pltpu.VMEM((NPTS * NP_NEIGH,), jnp.int32),
            pltpu.VMEM((CHUNK,), jnp.float32),
            pltpu.VMEM((CHUNK,), jnp.float32),
            pltpu.VMEM((CHUNK,), jnp.float32),
            pltpu.SemaphoreType.DMA,
            pltpu.SemaphoreType.DMA,
            pltpu.SemaphoreType.DMA,
        ],
    )
    outp = run(f1, f2, iflat)
    return outp.reshape(1, C, NPTS)
